# Initial kernel scaffold; baseline (speedup 1.0000x reference)
import jax
import jax.numpy as jnp
from jax.experimental import pallas as pl


def kernel(x, edge_index, batch, W1, a_s1, a_d1, b1, W2, a_s2, a_d2, b2, W3, a_s3, a_d3, b3, Wh, bh):
    raise NotImplementedError("write your pallas kernel here")


# speedup vs baseline: 15.1036x; 15.1036x over previous
import functools

import jax
import jax.numpy as jnp
from jax import lax
from jax.experimental import pallas as pl
from jax.experimental.pallas import tpu as pltpu
from jax.experimental.pallas import tpu_sc as plsc

N = 10000
E = 320000
D = 128
H = 128
A = 32
G = 16

NC = 2
NS = 16
NW = NC * NS
B = 128
EW = 10240
C = EW // B
E_PAD = NW * EW
ROWS_PER_TILE = 632
N_ACC = NS * ROWS_PER_TILE
DEN_ROWS = 80
OUT_ROWS = N_ACC + NS * DEN_ROWS


def _tc_first(x_ref, w_ref, as_ref, ad_ref, h_ref, aa_ref):
    x = x_ref[...]
    h = lax.dot_general(x, w_ref[...], (((1,), (1,)), ((), ())),
                        preferred_element_type=jnp.float32)
    h_ref[...] = h
    asrc = jnp.sum(h * as_ref[...][None, :], axis=1)
    adst = jnp.sum(h * ad_ref[...][None, :], axis=1)
    aa_ref[...] = jnp.stack([asrc, adst], axis=0)


def _den_from_acc(acc):
    dsum = acc[0, N_ACC:] + acc[1, N_ACC:]
    dsum = jnp.sum(dsum.reshape(NS, DEN_ROWS, H), axis=0)
    return dsum.reshape(DEN_ROWS * H)[:N]


def _tc_mid(acc_ref, hprev_ref, aa_ref, b_ref, w_ref, as_ref, ad_ref,
            h_ref, aaout_ref):
    aa = aa_ref[...]
    es = aa[0] + aa[1]
    es = jnp.exp(jnp.maximum(es, 0.2 * es))
    acc = acc_ref[...]
    num = acc[0, :N] + acc[1, :N] + es[:, None] * hprev_ref[...]
    den = _den_from_acc(acc) + es + 1e-16
    xn = jnp.maximum(num / den[:, None] + b_ref[...][None, :], 0.0)
    h = lax.dot_general(xn, w_ref[...], (((1,), (1,)), ((), ())),
                        preferred_element_type=jnp.float32)
    h_ref[...] = h
    asrc = jnp.sum(h * as_ref[...][None, :], axis=1)
    adst = jnp.sum(h * ad_ref[...][None, :], axis=1)
    aaout_ref[...] = jnp.stack([asrc, adst], axis=0)


def _tc_final(acc_ref, hprev_ref, aa_ref, b_ref, batch_ref, wh_ref,
              bh_ref, out_ref):
    aa = aa_ref[...]
    es = aa[0] + aa[1]
    es = jnp.exp(jnp.maximum(es, 0.2 * es))
    acc = acc_ref[...]
    num = acc[0, :N] + acc[1, :N] + es[:, None] * hprev_ref[...]
    den = _den_from_acc(acc) + es + 1e-16
    x3 = num / den[:, None] + b_ref[...][None, :]
    gids = lax.broadcasted_iota(jnp.int32, (G, N), 0)
    mask = (batch_ref[...][None, :] == gids).astype(jnp.float32)
    sums = lax.dot_general(mask, x3, (((1,), (0,)), ((), ())),
                           preferred_element_type=jnp.float32)
    counts = jnp.sum(mask, axis=1)
    pooled = sums / jnp.clip(counts, 1.0, None)[:, None]
    out = lax.dot_general(pooled, wh_ref[...], (((1,), (1,)), ((), ())),
                          preferred_element_type=jnp.float32)
    out_ref[...] = out + bh_ref[...][None, :]


def _sc_edge_body(h_hbm, aa_hbm, src_hbm, dst_hbm, out_hbm,
                  asrc_v, adst_v, den_v, src_c, dst_c, rows_v,
                  acc_sh, sem):
    c = lax.axis_index("c")
    s = lax.axis_index("s")
    wid = s * NC + c

    pltpu.sync_copy(aa_hbm.at[0], asrc_v)
    pltpu.sync_copy(aa_hbm.at[1], adst_v)

    zeros16 = jnp.zeros((16,), jnp.float32)

    def zden(r, _):
        for j in range(H // 16):
            den_v[r, pl.ds(j * 16, 16)] = zeros16
        return 0

    lax.fori_loop(0, DEN_ROWS, zden, 0)

    def zrow(r, _):
        for j in range(H // 16):
            rows_v[r, pl.ds(j * 16, 16)] = zeros16
        return 0

    lax.fori_loop(0, B, zrow, 0)
    row0 = s * ROWS_PER_TILE
    for j in range(ROWS_PER_TILE // B):
        pltpu.sync_copy(rows_v, acc_sh.at[pl.ds(row0 + j * B, B)])
    rem = ROWS_PER_TILE % B
    if rem:
        pltpu.sync_copy(
            rows_v.at[pl.ds(0, rem)],
            acc_sh.at[pl.ds(row0 + (ROWS_PER_TILE // B) * B, rem)])
    plsc.subcore_barrier()

    lanes = lax.iota(jnp.int32, 16)
    ebase = wid * EW

    def chunk(g, _):
        pltpu.sync_copy(src_hbm.at[wid, g], src_c)
        pltpu.sync_copy(dst_hbm.at[wid, g], dst_c)
        pltpu.async_copy(h_hbm.at[src_c], rows_v, sem).wait()

        def group(t, _):
            sv = src_c[pl.ds(t * 16, 16)]
            dv = dst_c[pl.ds(t * 16, 16)]
            e = (plsc.load_gather(asrc_v, [sv])
                 + plsc.load_gather(adst_v, [dv]))
            e = jnp.maximum(e, 0.2 * e)
            ex = jnp.exp(e)
            gid = ebase + g * B + t * 16 + lanes
            ex = jnp.where(gid < E, ex, 0.0)
            plsc.addupdate_scatter(
                den_v, [jnp.right_shift(dv, 7), jnp.bitwise_and(dv, 127)], ex)
            for rj in range(16):
                svec = jnp.full((16,), ex[rj], jnp.float32)
                r = t * 16 + rj
                for j in range(H // 16):
                    rows_v[r, pl.ds(j * 16, 16)] = (
                        rows_v[r, pl.ds(j * 16, 16)] * svec)
            return 0

        lax.fori_loop(0, B // 16, group, 0)
        pltpu.sync_copy(rows_v, acc_sh.at[dst_c], add=True)
        return 0

    lax.fori_loop(0, C, chunk, 0)

    pltpu.sync_copy(den_v,
                    out_hbm.at[c, pl.ds(N_ACC + s * DEN_ROWS, DEN_ROWS)])
    plsc.subcore_barrier()
    pltpu.sync_copy(acc_sh.at[pl.ds(row0, ROWS_PER_TILE)],
                    out_hbm.at[c, pl.ds(row0, ROWS_PER_TILE)])


@functools.cache
def _get_sc_edge():
    return pl.kernel(
        _sc_edge_body,
        out_type=jax.ShapeDtypeStruct((NC, OUT_ROWS, H), jnp.float32),
        mesh=plsc.VectorSubcoreMesh(core_axis_name="c", subcore_axis_name="s",
                                    num_cores=NC, num_subcores=NS),
        compiler_params=pltpu.CompilerParams(needs_layout_passes=False),
        scratch_types=[
            pltpu.VMEM((N,), jnp.float32),
            pltpu.VMEM((N,), jnp.float32),
            pltpu.VMEM((DEN_ROWS, H), jnp.float32),
            pltpu.VMEM((B,), jnp.int32),
            pltpu.VMEM((B,), jnp.int32),
            pltpu.VMEM((B, H), jnp.float32),
            pltpu.VMEM_SHARED((N_ACC, H), jnp.float32),
            pltpu.SemaphoreType.DMA,
        ],
    )


def kernel(x, edge_index, batch, W1, a_s1, a_d1, b1, W2, a_s2, a_d2, b2,
           W3, a_s3, a_d3, b3, Wh, bh):
    pad = E_PAD - E
    src = jnp.concatenate([edge_index[0], jnp.zeros((pad,), jnp.int32)])
    dst = jnp.concatenate([edge_index[1], jnp.zeros((pad,), jnp.int32)])
    src3 = src.reshape(NW, C, B)
    dst3 = dst.reshape(NW, C, B)

    h1, aa1 = pl.pallas_call(
        _tc_first,
        out_shape=[jax.ShapeDtypeStruct((N, H), jnp.float32),
                   jax.ShapeDtypeStruct((2, N), jnp.float32)],
    )(x, W1, a_s1, a_d1)

    sc_edge = _get_sc_edge()
    acc1 = sc_edge(h1, aa1, src3, dst3)

    h2, aa2 = pl.pallas_call(
        _tc_mid,
        out_shape=[jax.ShapeDtypeStruct((N, H), jnp.float32),
                   jax.ShapeDtypeStruct((2, N), jnp.float32)],
    )(acc1, h1, aa1, b1, W2, a_s2, a_d2)

    acc2 = sc_edge(h2, aa2, src3, dst3)

    h3, aa3 = pl.pallas_call(
        _tc_mid,
        out_shape=[jax.ShapeDtypeStruct((N, H), jnp.float32),
                   jax.ShapeDtypeStruct((2, N), jnp.float32)],
    )(acc2, h2, aa2, b2, W3, a_s3, a_d3)

    acc3 = sc_edge(h3, aa3, src3, dst3)

    out = pl.pallas_call(
        _tc_final,
        out_shape=jax.ShapeDtypeStruct((G, A), jnp.float32),
    )(acc3, h3, aa3, b3, batch, Wh, bh)

    return out


# speedup vs baseline: 20.5126x; 1.3581x over previous
import functools

import jax
import jax.numpy as jnp
from jax import lax
from jax.experimental import pallas as pl
from jax.experimental.pallas import tpu as pltpu
from jax.experimental.pallas import tpu_sc as plsc

N = 10000
E = 320000
D = 128
H = 128
A = 32
G = 16

NC = 2
NS = 16
NW = NC * NS
B = 128
EW = 10240
C = EW // B
E_PAD = NW * EW
HB = 64
ROWS_PER_TILE = 632
N_ACC = NS * ROWS_PER_TILE
DEN_ROWS = 80
OUT_ROWS = N_ACC + NS * DEN_ROWS


def _tc_first(x_ref, w_ref, as_ref, ad_ref, h_ref, aa_ref):
    x = x_ref[...]
    h = lax.dot_general(x, w_ref[...], (((1,), (1,)), ((), ())),
                        preferred_element_type=jnp.float32)
    h_ref[...] = h
    asrc = jnp.sum(h * as_ref[...][None, :], axis=1)
    adst = jnp.sum(h * ad_ref[...][None, :], axis=1)
    aa_ref[...] = jnp.stack([asrc, adst], axis=0)


def _den_from_acc(acc):
    dsum = acc[0, N_ACC:] + acc[1, N_ACC:]
    dsum = jnp.sum(dsum.reshape(NS, DEN_ROWS, H), axis=0)
    return dsum.reshape(DEN_ROWS * H)[:N]


def _tc_mid(acc_ref, hprev_ref, aa_ref, b_ref, w_ref, as_ref, ad_ref,
            h_ref, aaout_ref):
    aa = aa_ref[...]
    es = aa[0] + aa[1]
    es = jnp.exp(jnp.maximum(es, 0.2 * es))
    acc = acc_ref[...]
    num = acc[0, :N] + acc[1, :N] + es[:, None] * hprev_ref[...]
    den = _den_from_acc(acc) + es + 1e-16
    xn = jnp.maximum(num / den[:, None] + b_ref[...][None, :], 0.0)
    h = lax.dot_general(xn, w_ref[...], (((1,), (1,)), ((), ())),
                        preferred_element_type=jnp.float32)
    h_ref[...] = h
    asrc = jnp.sum(h * as_ref[...][None, :], axis=1)
    adst = jnp.sum(h * ad_ref[...][None, :], axis=1)
    aaout_ref[...] = jnp.stack([asrc, adst], axis=0)


def _tc_final(acc_ref, hprev_ref, aa_ref, b_ref, batch_ref, wh_ref,
              bh_ref, out_ref):
    aa = aa_ref[...]
    es = aa[0] + aa[1]
    es = jnp.exp(jnp.maximum(es, 0.2 * es))
    acc = acc_ref[...]
    num = acc[0, :N] + acc[1, :N] + es[:, None] * hprev_ref[...]
    den = _den_from_acc(acc) + es + 1e-16
    x3 = num / den[:, None] + b_ref[...][None, :]
    gids = lax.broadcasted_iota(jnp.int32, (G, N), 0)
    mask = (batch_ref[...][None, :] == gids).astype(jnp.float32)
    sums = lax.dot_general(mask, x3, (((1,), (0,)), ((), ())),
                           preferred_element_type=jnp.float32)
    counts = jnp.sum(mask, axis=1)
    pooled = sums / jnp.clip(counts, 1.0, None)[:, None]
    out = lax.dot_general(pooled, wh_ref[...], (((1,), (1,)), ((), ())),
                          preferred_element_type=jnp.float32)
    out_ref[...] = out + bh_ref[...][None, :]


def _sc_edge_body(h_hbm, aa_hbm, src_hbm, dst_hbm, out_hbm,
                  asrc_v, adst_v, den_v, src_i0, src_i1, dst_i0, dst_i1,
                  rows_v, acc_sh,
                  sem_g0, sem_g1, sem_s0, sem_s1, sem_i0, sem_i1):
    c = lax.axis_index("c")
    s = lax.axis_index("s")
    wid = s * NC + c

    pltpu.sync_copy(aa_hbm.at[0], asrc_v)
    pltpu.sync_copy(aa_hbm.at[1], adst_v)

    zeros16 = jnp.zeros((16,), jnp.float32)

    def zden(r, _):
        for j in range(H // 16):
            den_v[r, pl.ds(j * 16, 16)] = zeros16
        return 0

    lax.fori_loop(0, DEN_ROWS, zden, 0)

    def zrow(r, _):
        for j in range(H // 16):
            rows_v[r, pl.ds(j * 16, 16)] = zeros16
        return 0

    lax.fori_loop(0, B, zrow, 0)
    row0 = s * ROWS_PER_TILE
    for j in range(ROWS_PER_TILE // B):
        pltpu.sync_copy(rows_v, acc_sh.at[pl.ds(row0 + j * B, B)])
    rem = ROWS_PER_TILE % B
    if rem:
        pltpu.sync_copy(
            rows_v.at[pl.ds(0, rem)],
            acc_sh.at[pl.ds(row0 + (ROWS_PER_TILE // B) * B, rem)])
    plsc.subcore_barrier()

    lanes = lax.iota(jnp.int32, 16)
    ebase = wid * EW

    src_i = (src_i0, src_i1)
    dst_i = (dst_i0, dst_i1)
    sem_g = (sem_g0, sem_g1)
    sem_s = (sem_s0, sem_s1)
    sem_i = (sem_i0, sem_i1)
    rows = (rows_v.at[pl.ds(0, HB)], rows_v.at[pl.ds(HB, HB)])
    NSTEP = 2 * C

    def idx_start(k_chunk, k_off, u):
        pltpu.async_copy(src_hbm.at[wid, k_chunk, pl.ds(k_off, HB)],
                         src_i[u], sem_i[u])
        pltpu.async_copy(dst_hbm.at[wid, k_chunk, pl.ds(k_off, HB)],
                         dst_i[u], sem_i[u])

    def idx_wait(u):
        pltpu.make_async_copy(src_hbm.at[wid, 0, pl.ds(0, HB)],
                              src_i[u], sem_i[u]).wait()
        pltpu.make_async_copy(dst_hbm.at[wid, 0, pl.ds(0, HB)],
                              dst_i[u], sem_i[u]).wait()

    def gather_start(u):
        pltpu.async_copy(h_hbm.at[src_i[u]], rows[u], sem_g[u])

    def gather_wait(u):
        pltpu.make_async_copy(h_hbm.at[src_i[u]], rows[u], sem_g[u]).wait()

    def scatter_start(u):
        pltpu.async_copy(rows[u], acc_sh.at[dst_i[u]], sem_s[u], add=True)

    def scatter_wait(u):
        pltpu.make_async_copy(rows[u], acc_sh.at[dst_i[u]], sem_s[u]).wait()

    def compute(k, u):
        base = u * HB
        for t in range(HB // 16):
            sv = src_i[u][pl.ds(t * 16, 16)]
            dv = dst_i[u][pl.ds(t * 16, 16)]
            e = (plsc.load_gather(asrc_v, [sv])
                 + plsc.load_gather(adst_v, [dv]))
            e = jnp.maximum(e, 0.2 * e)
            ex = jnp.exp(e)
            gid = ebase + k * HB + t * 16 + lanes
            ex = jnp.where(gid < E, ex, 0.0)
            plsc.addupdate_scatter(
                den_v, [jnp.right_shift(dv, 7), jnp.bitwise_and(dv, 127)], ex)
            for rj in range(16):
                svec = jnp.full((16,), ex[rj], jnp.float32)
                r = base + t * 16 + rj
                for j in range(H // 16):
                    rows_v[r, pl.ds(j * 16, 16)] = (
                        rows_v[r, pl.ds(j * 16, 16)] * svec)

    idx_start(0, 0, 0)
    idx_start(0, HB, 1)
    idx_wait(0)
    gather_start(0)

    def pair(j, _):
        @pl.when(j >= 1)
        def _():
            scatter_wait(1)

        idx_wait(1)
        gather_start(1)
        gather_wait(0)
        compute(2 * j, 0)
        scatter_start(0)

        @pl.when(j < C - 1)
        def _():
            idx_start(j + 1, 0, 0)

        scatter_wait(0)

        @pl.when(j < C - 1)
        def _():
            idx_wait(0)
            gather_start(0)

        gather_wait(1)
        compute(2 * j + 1, 1)
        scatter_start(1)

        @pl.when(j < C - 1)
        def _():
            idx_start(j + 1, HB, 1)

        return 0

    lax.fori_loop(0, C, pair, 0)
    scatter_wait(1)

    pltpu.sync_copy(den_v,
                    out_hbm.at[c, pl.ds(N_ACC + s * DEN_ROWS, DEN_ROWS)])
    plsc.subcore_barrier()
    pltpu.sync_copy(acc_sh.at[pl.ds(row0, ROWS_PER_TILE)],
                    out_hbm.at[c, pl.ds(row0, ROWS_PER_TILE)])


@functools.cache
def _get_sc_edge():
    return pl.kernel(
        _sc_edge_body,
        out_type=jax.ShapeDtypeStruct((NC, OUT_ROWS, H), jnp.float32),
        mesh=plsc.VectorSubcoreMesh(core_axis_name="c", subcore_axis_name="s",
                                    num_cores=NC, num_subcores=NS),
        compiler_params=pltpu.CompilerParams(needs_layout_passes=False),
        scratch_types=[
            pltpu.VMEM((N,), jnp.float32),
            pltpu.VMEM((N,), jnp.float32),
            pltpu.VMEM((DEN_ROWS, H), jnp.float32),
            pltpu.VMEM((HB,), jnp.int32),
            pltpu.VMEM((HB,), jnp.int32),
            pltpu.VMEM((HB,), jnp.int32),
            pltpu.VMEM((HB,), jnp.int32),
            pltpu.VMEM((B, H), jnp.float32),
            pltpu.VMEM_SHARED((N_ACC, H), jnp.float32),
            pltpu.SemaphoreType.DMA,
            pltpu.SemaphoreType.DMA,
            pltpu.SemaphoreType.DMA,
            pltpu.SemaphoreType.DMA,
            pltpu.SemaphoreType.DMA,
            pltpu.SemaphoreType.DMA,
        ],
    )


def kernel(x, edge_index, batch, W1, a_s1, a_d1, b1, W2, a_s2, a_d2, b2,
           W3, a_s3, a_d3, b3, Wh, bh):
    pad = E_PAD - E
    src = jnp.concatenate([edge_index[0], jnp.zeros((pad,), jnp.int32)])
    dst = jnp.concatenate([edge_index[1], jnp.zeros((pad,), jnp.int32)])
    src3 = src.reshape(NW, C, B)
    dst3 = dst.reshape(NW, C, B)

    h1, aa1 = pl.pallas_call(
        _tc_first,
        out_shape=[jax.ShapeDtypeStruct((N, H), jnp.float32),
                   jax.ShapeDtypeStruct((2, N), jnp.float32)],
    )(x, W1, a_s1, a_d1)

    sc_edge = _get_sc_edge()
    acc1 = sc_edge(h1, aa1, src3, dst3)

    h2, aa2 = pl.pallas_call(
        _tc_mid,
        out_shape=[jax.ShapeDtypeStruct((N, H), jnp.float32),
                   jax.ShapeDtypeStruct((2, N), jnp.float32)],
    )(acc1, h1, aa1, b1, W2, a_s2, a_d2)

    acc2 = sc_edge(h2, aa2, src3, dst3)

    h3, aa3 = pl.pallas_call(
        _tc_mid,
        out_shape=[jax.ShapeDtypeStruct((N, H), jnp.float32),
                   jax.ShapeDtypeStruct((2, N), jnp.float32)],
    )(acc2, h2, aa2, b2, W3, a_s3, a_d3)

    acc3 = sc_edge(h3, aa3, src3, dst3)

    out = pl.pallas_call(
        _tc_final,
        out_shape=jax.ShapeDtypeStruct((G, A), jnp.float32),
    )(acc3, h3, aa3, b3, batch, Wh, bh)

    return out
